# baseline (speedup 1.0000x reference)
import functools

import jax
import jax.numpy as jnp
from jax import lax
from jax.experimental import pallas as pl
from jax.experimental.pallas import tpu as pltpu, tpu_sc as plsc

NUM_SAMPLES = 128
B = 32
D = 4096
SIGMA = 0.1

K = 2048
LO = -9.0
HI = 9.0
SCALE = K / (HI - LO)

L = 16
NC = 2
UNROLL = 8


def _rank_mean_kernel(x_hbm, g_hbm, out_hbm, xs_v, g0a_v, g0b_v, g1a_v,
                      g1b_v, ba_v, bb_v, h_v, vala_v, valb_v, acc_v,
                      s0a, s0b, s1a, s1b):
    wid = lax.axis_index("s") * NC + lax.axis_index("c")

    pltpu.sync_copy(x_hbm.at[wid], xs_v)

    @plsc.parallel_loop(0, D // L, unroll=UNROLL)
    def _init(i):
        sl = pl.ds(i * L, L)
        xs_v[sl] = (xs_v[sl] - LO) * SCALE
        acc_v[sl] = jnp.zeros((L,), jnp.float32)

    @plsc.parallel_loop(0, K // L, unroll=UNROLL)
    def _inith(i):
        sl = pl.ds(i * L, L)
        h_v[sl] = jnp.zeros((L,), jnp.int32)

    def process_pair(ga_v, gb_v, first):
        @plsc.parallel_loop(0, D // L, unroll=UNROLL)
        def _pass12(i):
            sl = pl.ds(i * L, L)
            bia_old = ba_v[sl]
            bib_old = bb_v[sl]
            if not first:
                ra = plsc.load_gather(vala_v, [bia_old])
                rb = plsc.load_gather(valb_v, [bib_old])
                plsc.addupdate(acc_v.at[sl], ra + rb)
            xs = xs_v[sl]
            ta = xs + ga_v[sl] * (SIGMA * SCALE)
            tb = xs + gb_v[sl] * (SIGMA * SCALE)
            ta = jnp.minimum(jnp.maximum(ta, 0.0), K - 1.0)
            tb = jnp.minimum(jnp.maximum(tb, 0.0), K - 1.0)
            bia = ta.astype(jnp.int32)
            bib = tb.astype(jnp.int32)
            ba_v[sl] = bia
            bb_v[sl] = bib
            plsc.addupdate_scatter(h_v, [bia], jnp.ones((L,), jnp.int32))
            plsc.addupdate_scatter(h_v, [bib],
                                   jnp.full((L,), 65536, jnp.int32))

        @plsc.parallel_loop(0, K // L, unroll=UNROLL,
                            carry=(jnp.int32(0), jnp.int32(0)))
        def _cum(i, carry):
            ca, cb = carry
            sl = pl.ds(i * L, L)
            h = h_v[sl]
            h_v[sl] = jnp.zeros((L,), jnp.int32)
            ha = jnp.bitwise_and(h, 0xFFFF)
            hb = lax.shift_right_logical(h, 16)
            inca = plsc.cumsum(ha) + ca
            incb = plsc.cumsum(hb) + cb
            haf = ha.astype(jnp.float32)
            hbf = hb.astype(jnp.float32)
            vala_v[sl] = (inca - ha).astype(jnp.float32) + (haf - 1.0) * 0.5
            valb_v[sl] = (incb - hb).astype(jnp.float32) + (hbf - 1.0) * 0.5
            return (ca + jnp.sum(ha), cb + jnp.sum(hb))

    def flush_pair():
        @plsc.parallel_loop(0, D // L, unroll=UNROLL)
        def _pass2(i):
            sl = pl.ds(i * L, L)
            ra = plsc.load_gather(vala_v, [ba_v[sl]])
            rb = plsc.load_gather(valb_v, [bb_v[sl]])
            plsc.addupdate(acc_v.at[sl], ra + rb)

    def row(s):
        return s * B + wid

    def fetch(pair, ga, gb, sa, sb):
        pltpu.async_copy(g_hbm.at[row(pair * 2)], ga, sa)
        pltpu.async_copy(g_hbm.at[row(pair * 2 + 1)], gb, sb)

    def wait(pair, ga, gb, sa, sb):
        pltpu.make_async_copy(g_hbm.at[row(pair * 2)], ga, sa).wait()
        pltpu.make_async_copy(g_hbm.at[row(pair * 2 + 1)], gb, sb).wait()

    NP = NUM_SAMPLES // 2
    fetch(0, g0a_v, g0b_v, s0a, s0b)
    fetch(1, g1a_v, g1b_v, s1a, s1b)
    wait(0, g0a_v, g0b_v, s0a, s0b)
    process_pair(g0a_v, g0b_v, first=True)
    fetch(2, g0a_v, g0b_v, s0a, s0b)

    def pair2_body(q, _):
        p1 = q * 2 + 1
        wait(p1, g1a_v, g1b_v, s1a, s1b)
        process_pair(g1a_v, g1b_v, first=False)
        fetch(p1 + 2, g1a_v, g1b_v, s1a, s1b)
        wait(p1 + 1, g0a_v, g0b_v, s0a, s0b)
        process_pair(g0a_v, g0b_v, first=False)
        nxt = jnp.minimum(p1 + 3, NP - 1)
        fetch(nxt, g0a_v, g0b_v, s0a, s0b)
        return 0

    lax.fori_loop(0, NP // 2 - 1, pair2_body, 0)
    wait(NP - 1, g1a_v, g1b_v, s1a, s1b)
    process_pair(g1a_v, g1b_v, first=False)
    flush_pair()
    wait(NP - 1, g0a_v, g0b_v, s0a, s0b)

    @plsc.parallel_loop(0, D // L, unroll=UNROLL)
    def _fin(i):
        sl = pl.ds(i * L, L)
        acc_v[sl] = acc_v[sl] * (1.0 / NUM_SAMPLES)
    pltpu.sync_copy(acc_v, out_hbm.at[wid])


def kernel(inputs, gumbel_noise):
    noise2d = gumbel_noise.reshape(NUM_SAMPLES * B, D)
    mesh = plsc.VectorSubcoreMesh(core_axis_name="c", subcore_axis_name="s")
    run = functools.partial(
        pl.kernel,
        out_type=jax.ShapeDtypeStruct((B, D), jnp.float32),
        mesh=mesh,
        compiler_params=pltpu.CompilerParams(needs_layout_passes=False),
        scratch_types=[
            pltpu.VMEM((D,), jnp.float32),
            pltpu.VMEM((D,), jnp.float32),
            pltpu.VMEM((D,), jnp.float32),
            pltpu.VMEM((D,), jnp.float32),
            pltpu.VMEM((D,), jnp.float32),
            pltpu.VMEM((D,), jnp.int32),
            pltpu.VMEM((D,), jnp.int32),
            pltpu.VMEM((K,), jnp.int32),
            pltpu.VMEM((K,), jnp.float32),
            pltpu.VMEM((K,), jnp.float32),
            pltpu.VMEM((D,), jnp.float32),
            pltpu.SemaphoreType.DMA,
            pltpu.SemaphoreType.DMA,
            pltpu.SemaphoreType.DMA,
            pltpu.SemaphoreType.DMA,
        ],
    )(_rank_mean_kernel)
    return run(inputs, noise2d)


# speedup vs baseline: 1.0066x; 1.0066x over previous
import functools

import jax
import jax.numpy as jnp
from jax import lax
from jax.experimental import pallas as pl
from jax.experimental.pallas import tpu as pltpu, tpu_sc as plsc

NUM_SAMPLES = 128
B = 32
D = 4096
SIGMA = 0.1

K = 2048
LO = -9.0
HI = 9.0
SCALE = K / (HI - LO)

L = 16
NC = 2
UNROLL = 8


def _rank_mean_kernel(x_hbm, g_hbm, out_hbm, xs_v, g0a_v, g0b_v, g1a_v,
                      g1b_v, ba_v, bb_v, h_v, vala_v, valb_v, acc_v,
                      s0a, s0b, s1a, s1b):
    wid = lax.axis_index("s") * NC + lax.axis_index("c")

    pltpu.sync_copy(x_hbm.at[wid], xs_v)

    @plsc.parallel_loop(0, D // L, unroll=UNROLL)
    def _init(i):
        sl = pl.ds(i * L, L)
        xs_v[sl] = (xs_v[sl] - LO) * SCALE
        acc_v[sl] = jnp.zeros((L,), jnp.float32)

    @plsc.parallel_loop(0, K // L, unroll=UNROLL)
    def _inith(i):
        sl = pl.ds(i * L, L)
        h_v[sl] = jnp.zeros((L,), jnp.int32)

    def process_pair(ga_v, gb_v, first):
        @plsc.parallel_loop(0, D // L, unroll=4)
        def _pass12(i):
            sl = pl.ds(i * L, L)
            bia_old = ba_v[sl]
            bib_old = bb_v[sl]
            if not first:
                ra = plsc.load_gather(vala_v, [bia_old])
                rb = plsc.load_gather(valb_v, [bib_old])
                plsc.addupdate(acc_v.at[sl], ra + rb)
            xs = xs_v[sl]
            ta = xs + ga_v[sl] * (SIGMA * SCALE)
            tb = xs + gb_v[sl] * (SIGMA * SCALE)
            ta = jnp.minimum(jnp.maximum(ta, 0.0), K - 1.0)
            tb = jnp.minimum(jnp.maximum(tb, 0.0), K - 1.0)
            bia = ta.astype(jnp.int32)
            bib = tb.astype(jnp.int32)
            ba_v[sl] = bia
            bb_v[sl] = bib
            plsc.addupdate_scatter(h_v, [bia], jnp.ones((L,), jnp.int32))
            plsc.addupdate_scatter(h_v, [bib],
                                   jnp.full((L,), 65536, jnp.int32))

        @plsc.parallel_loop(0, K // L, unroll=UNROLL,
                            carry=(jnp.int32(0), jnp.int32(0)))
        def _cum(i, carry):
            ca, cb = carry
            sl = pl.ds(i * L, L)
            h = h_v[sl]
            h_v[sl] = jnp.zeros((L,), jnp.int32)
            ha = jnp.bitwise_and(h, 0xFFFF)
            hb = lax.shift_right_logical(h, 16)
            inca = plsc.cumsum(ha) + ca
            incb = plsc.cumsum(hb) + cb
            haf = ha.astype(jnp.float32)
            hbf = hb.astype(jnp.float32)
            vala_v[sl] = (inca - ha).astype(jnp.float32) + (haf - 1.0) * 0.5
            valb_v[sl] = (incb - hb).astype(jnp.float32) + (hbf - 1.0) * 0.5
            return (ca + jnp.sum(ha), cb + jnp.sum(hb))

    def flush_pair():
        @plsc.parallel_loop(0, D // L, unroll=UNROLL)
        def _pass2(i):
            sl = pl.ds(i * L, L)
            ra = plsc.load_gather(vala_v, [ba_v[sl]])
            rb = plsc.load_gather(valb_v, [bb_v[sl]])
            plsc.addupdate(acc_v.at[sl], ra + rb)

    def row(s):
        return s * B + wid

    def fetch(pair, ga, gb, sa, sb):
        pltpu.async_copy(g_hbm.at[row(pair * 2)], ga, sa)
        pltpu.async_copy(g_hbm.at[row(pair * 2 + 1)], gb, sb)

    def wait(pair, ga, gb, sa, sb):
        pltpu.make_async_copy(g_hbm.at[row(pair * 2)], ga, sa).wait()
        pltpu.make_async_copy(g_hbm.at[row(pair * 2 + 1)], gb, sb).wait()

    NP = NUM_SAMPLES // 2
    fetch(0, g0a_v, g0b_v, s0a, s0b)
    fetch(1, g1a_v, g1b_v, s1a, s1b)
    wait(0, g0a_v, g0b_v, s0a, s0b)
    process_pair(g0a_v, g0b_v, first=True)
    fetch(2, g0a_v, g0b_v, s0a, s0b)

    def pair2_body(q, _):
        p1 = q * 2 + 1
        wait(p1, g1a_v, g1b_v, s1a, s1b)
        process_pair(g1a_v, g1b_v, first=False)
        fetch(p1 + 2, g1a_v, g1b_v, s1a, s1b)
        wait(p1 + 1, g0a_v, g0b_v, s0a, s0b)
        process_pair(g0a_v, g0b_v, first=False)
        nxt = jnp.minimum(p1 + 3, NP - 1)
        fetch(nxt, g0a_v, g0b_v, s0a, s0b)
        return 0

    lax.fori_loop(0, NP // 2 - 1, pair2_body, 0)
    wait(NP - 1, g1a_v, g1b_v, s1a, s1b)
    process_pair(g1a_v, g1b_v, first=False)
    flush_pair()
    wait(NP - 1, g0a_v, g0b_v, s0a, s0b)

    @plsc.parallel_loop(0, D // L, unroll=UNROLL)
    def _fin(i):
        sl = pl.ds(i * L, L)
        acc_v[sl] = acc_v[sl] * (1.0 / NUM_SAMPLES)
    pltpu.sync_copy(acc_v, out_hbm.at[wid])


def kernel(inputs, gumbel_noise):
    noise2d = gumbel_noise.reshape(NUM_SAMPLES * B, D)
    mesh = plsc.VectorSubcoreMesh(core_axis_name="c", subcore_axis_name="s")
    run = functools.partial(
        pl.kernel,
        out_type=jax.ShapeDtypeStruct((B, D), jnp.float32),
        mesh=mesh,
        compiler_params=pltpu.CompilerParams(needs_layout_passes=False),
        scratch_types=[
            pltpu.VMEM((D,), jnp.float32),
            pltpu.VMEM((D,), jnp.float32),
            pltpu.VMEM((D,), jnp.float32),
            pltpu.VMEM((D,), jnp.float32),
            pltpu.VMEM((D,), jnp.float32),
            pltpu.VMEM((D,), jnp.int32),
            pltpu.VMEM((D,), jnp.int32),
            pltpu.VMEM((K,), jnp.int32),
            pltpu.VMEM((K,), jnp.float32),
            pltpu.VMEM((K,), jnp.float32),
            pltpu.VMEM((D,), jnp.float32),
            pltpu.SemaphoreType.DMA,
            pltpu.SemaphoreType.DMA,
            pltpu.SemaphoreType.DMA,
            pltpu.SemaphoreType.DMA,
        ],
    )(_rank_mean_kernel)
    return run(inputs, noise2d)


# speedup vs baseline: 1.0184x; 1.0117x over previous
import functools

import jax
import jax.numpy as jnp
from jax import lax
from jax.experimental import pallas as pl
from jax.experimental.pallas import tpu as pltpu, tpu_sc as plsc

NUM_SAMPLES = 128
B = 32
D = 4096
SIGMA = 0.1

K = 2048
LO = -9.0
HI = 9.0
SCALE = K / (HI - LO)

L = 16
NC = 2
UNROLL = 8


def _rank_mean_kernel(x_hbm, g_hbm, out_hbm, xs_v, g0a_v, g0b_v, g1a_v,
                      g1b_v, ba_v, bb_v, h_v, vala_v, valb_v, acc_v,
                      s0a, s0b, s1a, s1b):
    wid = lax.axis_index("s") * NC + lax.axis_index("c")

    pltpu.sync_copy(x_hbm.at[wid], xs_v)

    @plsc.parallel_loop(0, D // L, unroll=UNROLL)
    def _init(i):
        sl = pl.ds(i * L, L)
        xs_v[sl] = (xs_v[sl] - LO) * SCALE
        acc_v[sl] = jnp.zeros((L,), jnp.float32)

    @plsc.parallel_loop(0, K // L, unroll=UNROLL)
    def _inith(i):
        sl = pl.ds(i * L, L)
        h_v[sl] = jnp.zeros((L,), jnp.int32)

    def process_pair(ga_v, gb_v):
        @plsc.parallel_loop(0, D // L, unroll=UNROLL)
        def _pass1(i):
            sl = pl.ds(i * L, L)
            xs = xs_v[sl]
            ta = xs + ga_v[sl] * (SIGMA * SCALE)
            tb = xs + gb_v[sl] * (SIGMA * SCALE)
            ta = jnp.minimum(jnp.maximum(ta, 0.0), K - 1.0)
            tb = jnp.minimum(jnp.maximum(tb, 0.0), K - 1.0)
            bia = ta.astype(jnp.int32)
            bib = tb.astype(jnp.int32)
            ba_v[sl] = bia
            bb_v[sl] = bib
            plsc.addupdate_scatter(h_v, [bia], jnp.ones((L,), jnp.int32))
            plsc.addupdate_scatter(h_v, [bib],
                                   jnp.full((L,), 65536, jnp.int32))

        @plsc.parallel_loop(0, K // L, unroll=UNROLL,
                            carry=(jnp.int32(0), jnp.int32(0)))
        def _cum(i, carry):
            ca, cb = carry
            sl = pl.ds(i * L, L)
            h = h_v[sl]
            h_v[sl] = jnp.zeros((L,), jnp.int32)
            ha = jnp.bitwise_and(h, 0xFFFF)
            hb = lax.shift_right_logical(h, 16)
            inca = plsc.cumsum(ha) + ca
            incb = plsc.cumsum(hb) + cb
            haf = ha.astype(jnp.float32)
            hbf = hb.astype(jnp.float32)
            vala_v[sl] = (inca - ha).astype(jnp.float32) + (haf - 1.0) * 0.5
            valb_v[sl] = (incb - hb).astype(jnp.float32) + (hbf - 1.0) * 0.5
            return (ca + jnp.sum(ha), cb + jnp.sum(hb))

        @plsc.parallel_loop(0, D // L, unroll=UNROLL)
        def _pass2(i):
            sl = pl.ds(i * L, L)
            ra = plsc.load_gather(vala_v, [ba_v[sl]])
            rb = plsc.load_gather(valb_v, [bb_v[sl]])
            plsc.addupdate(acc_v.at[sl], ra + rb)

    def row(s):
        return s * B + wid

    def fetch(pair, ga, gb, sa, sb):
        pltpu.async_copy(g_hbm.at[row(pair * 2)], ga, sa)
        pltpu.async_copy(g_hbm.at[row(pair * 2 + 1)], gb, sb)

    def wait(pair, ga, gb, sa, sb):
        pltpu.make_async_copy(g_hbm.at[row(pair * 2)], ga, sa).wait()
        pltpu.make_async_copy(g_hbm.at[row(pair * 2 + 1)], gb, sb).wait()

    NP = NUM_SAMPLES // 2
    fetch(0, g0a_v, g0b_v, s0a, s0b)

    def pair2_body(q, _):
        p0 = q * 2
        fetch(p0 + 1, g1a_v, g1b_v, s1a, s1b)
        wait(p0, g0a_v, g0b_v, s0a, s0b)
        process_pair(g0a_v, g0b_v)
        nxt = jnp.minimum(p0 + 2, NP - 1)
        fetch(nxt, g0a_v, g0b_v, s0a, s0b)
        wait(p0 + 1, g1a_v, g1b_v, s1a, s1b)
        process_pair(g1a_v, g1b_v)
        return 0

    lax.fori_loop(0, NP // 2, pair2_body, 0)
    wait(NP - 1, g0a_v, g0b_v, s0a, s0b)

    @plsc.parallel_loop(0, D // L, unroll=UNROLL)
    def _fin(i):
        sl = pl.ds(i * L, L)
        acc_v[sl] = acc_v[sl] * (1.0 / NUM_SAMPLES)
    pltpu.sync_copy(acc_v, out_hbm.at[wid])


def kernel(inputs, gumbel_noise):
    noise2d = gumbel_noise.reshape(NUM_SAMPLES * B, D)
    mesh = plsc.VectorSubcoreMesh(core_axis_name="c", subcore_axis_name="s")
    run = functools.partial(
        pl.kernel,
        out_type=jax.ShapeDtypeStruct((B, D), jnp.float32),
        mesh=mesh,
        compiler_params=pltpu.CompilerParams(needs_layout_passes=False),
        scratch_types=[
            pltpu.VMEM((D,), jnp.float32),
            pltpu.VMEM((D,), jnp.float32),
            pltpu.VMEM((D,), jnp.float32),
            pltpu.VMEM((D,), jnp.float32),
            pltpu.VMEM((D,), jnp.float32),
            pltpu.VMEM((D,), jnp.int32),
            pltpu.VMEM((D,), jnp.int32),
            pltpu.VMEM((K,), jnp.int32),
            pltpu.VMEM((K,), jnp.float32),
            pltpu.VMEM((K,), jnp.float32),
            pltpu.VMEM((D,), jnp.float32),
            pltpu.SemaphoreType.DMA,
            pltpu.SemaphoreType.DMA,
            pltpu.SemaphoreType.DMA,
            pltpu.SemaphoreType.DMA,
        ],
    )(_rank_mean_kernel)
    return run(inputs, noise2d)


# speedup vs baseline: 1.0965x; 1.0767x over previous
import functools

import jax
import jax.numpy as jnp
from jax import lax
from jax.experimental import pallas as pl
from jax.experimental.pallas import tpu as pltpu, tpu_sc as plsc

NUM_SAMPLES = 128
B = 32
D = 4096
SIGMA = 0.1

K = 2048
LO = -9.0
HI = 9.0
SCALE = K / (HI - LO)

L = 16
NC = 2
UNROLL = 8


def _rank_mean_kernel(x_hbm, g_hbm, out_hbm, xs_v, g0a_v, g0b_v, g1a_v,
                      g1b_v, ba_v, h_v, vala_v, valb_v, acc_v,
                      s0a, s0b, s1a, s1b):
    wid = lax.axis_index("s") * NC + lax.axis_index("c")

    pltpu.sync_copy(x_hbm.at[wid], xs_v)

    @plsc.parallel_loop(0, D // L, unroll=UNROLL)
    def _init(i):
        sl = pl.ds(i * L, L)
        xs_v[sl] = (xs_v[sl] - LO) * SCALE
        acc_v[sl] = jnp.zeros((L,), jnp.float32)

    @plsc.parallel_loop(0, K // L, unroll=UNROLL)
    def _inith(i):
        sl = pl.ds(i * L, L)
        h_v[sl] = jnp.zeros((L,), jnp.int32)

    def process_pair(ga_v, gb_v):
        @plsc.parallel_loop(0, D // L, unroll=UNROLL)
        def _pass1(i):
            sl = pl.ds(i * L, L)
            xs = xs_v[sl]
            ta = xs + ga_v[sl] * (SIGMA * SCALE)
            tb = xs + gb_v[sl] * (SIGMA * SCALE)
            ta = jnp.minimum(jnp.maximum(ta, 0.0), K - 1.0)
            tb = jnp.minimum(jnp.maximum(tb, 0.0), K - 1.0)
            bia = ta.astype(jnp.int32)
            bib = tb.astype(jnp.int32)
            ba_v[sl] = jnp.bitwise_or(bia, lax.shift_left(bib, 16))
            plsc.addupdate_scatter(h_v, [bia], jnp.ones((L,), jnp.int32))
            plsc.addupdate_scatter(h_v, [bib],
                                   jnp.full((L,), 65536, jnp.int32))

        @plsc.parallel_loop(0, K // L, unroll=UNROLL,
                            carry=(jnp.int32(0), jnp.int32(0)))
        def _cum(i, carry):
            ca, cb = carry
            sl = pl.ds(i * L, L)
            h = h_v[sl]
            h_v[sl] = jnp.zeros((L,), jnp.int32)
            ha = jnp.bitwise_and(h, 0xFFFF)
            hb = lax.shift_right_logical(h, 16)
            inca = plsc.cumsum(ha) + ca
            incb = plsc.cumsum(hb) + cb
            haf = ha.astype(jnp.float32)
            hbf = hb.astype(jnp.float32)
            vala_v[sl] = (inca - ha).astype(jnp.float32) + (haf - 1.0) * 0.5
            valb_v[sl] = (incb - hb).astype(jnp.float32) + (hbf - 1.0) * 0.5
            return (ca + jnp.sum(ha), cb + jnp.sum(hb))

        @plsc.parallel_loop(0, D // L, unroll=UNROLL)
        def _pass2(i):
            sl = pl.ds(i * L, L)
            w = ba_v[sl]
            ra = plsc.load_gather(vala_v, [jnp.bitwise_and(w, 0xFFFF)])
            rb = plsc.load_gather(valb_v, [lax.shift_right_logical(w, 16)])
            plsc.addupdate(acc_v.at[sl], ra + rb)

    def row(s):
        return s * B + wid

    def fetch(pair, ga, gb, sa, sb):
        pltpu.async_copy(g_hbm.at[row(pair * 2)], ga, sa)
        pltpu.async_copy(g_hbm.at[row(pair * 2 + 1)], gb, sb)

    def wait(pair, ga, gb, sa, sb):
        pltpu.make_async_copy(g_hbm.at[row(pair * 2)], ga, sa).wait()
        pltpu.make_async_copy(g_hbm.at[row(pair * 2 + 1)], gb, sb).wait()

    NP = NUM_SAMPLES // 2
    fetch(0, g0a_v, g0b_v, s0a, s0b)

    def pair2_body(q, _):
        p0 = q * 2
        fetch(p0 + 1, g1a_v, g1b_v, s1a, s1b)
        wait(p0, g0a_v, g0b_v, s0a, s0b)
        process_pair(g0a_v, g0b_v)
        nxt = jnp.minimum(p0 + 2, NP - 1)
        fetch(nxt, g0a_v, g0b_v, s0a, s0b)
        wait(p0 + 1, g1a_v, g1b_v, s1a, s1b)
        process_pair(g1a_v, g1b_v)
        return 0

    lax.fori_loop(0, NP // 2, pair2_body, 0)
    wait(NP - 1, g0a_v, g0b_v, s0a, s0b)

    @plsc.parallel_loop(0, D // L, unroll=UNROLL)
    def _fin(i):
        sl = pl.ds(i * L, L)
        acc_v[sl] = acc_v[sl] * (1.0 / NUM_SAMPLES)
    pltpu.sync_copy(acc_v, out_hbm.at[wid])


def kernel(inputs, gumbel_noise):
    noise2d = gumbel_noise.reshape(NUM_SAMPLES * B, D)
    mesh = plsc.VectorSubcoreMesh(core_axis_name="c", subcore_axis_name="s")
    run = functools.partial(
        pl.kernel,
        out_type=jax.ShapeDtypeStruct((B, D), jnp.float32),
        mesh=mesh,
        compiler_params=pltpu.CompilerParams(needs_layout_passes=False),
        scratch_types=[
            pltpu.VMEM((D,), jnp.float32),
            pltpu.VMEM((D,), jnp.float32),
            pltpu.VMEM((D,), jnp.float32),
            pltpu.VMEM((D,), jnp.float32),
            pltpu.VMEM((D,), jnp.float32),
            pltpu.VMEM((D,), jnp.int32),
            pltpu.VMEM((K,), jnp.int32),
            pltpu.VMEM((K,), jnp.float32),
            pltpu.VMEM((K,), jnp.float32),
            pltpu.VMEM((D,), jnp.float32),
            pltpu.SemaphoreType.DMA,
            pltpu.SemaphoreType.DMA,
            pltpu.SemaphoreType.DMA,
            pltpu.SemaphoreType.DMA,
        ],
    )(_rank_mean_kernel)
    return run(inputs, noise2d)


# speedup vs baseline: 1.1030x; 1.0059x over previous
import functools

import jax
import jax.numpy as jnp
from jax import lax
from jax.experimental import pallas as pl
from jax.experimental.pallas import tpu as pltpu, tpu_sc as plsc

NUM_SAMPLES = 128
B = 32
D = 4096
SIGMA = 0.1

K = 2048
LO = -9.0
HI = 9.0
SCALE = K / (HI - LO)

L = 16
NC = 2
UNROLL = 8


def _rank_mean_kernel(x_hbm, g_hbm, out_hbm, xs_v, g0a_v, g0b_v, g1a_v,
                      g1b_v, ba_v, h_v, vala_v, valb_v, acc_v,
                      s0a, s0b, s1a, s1b):
    wid = lax.axis_index("s") * NC + lax.axis_index("c")

    pltpu.sync_copy(x_hbm.at[wid], xs_v)

    @plsc.parallel_loop(0, D // L, unroll=UNROLL)
    def _init(i):
        sl = pl.ds(i * L, L)
        xs_v[sl] = (xs_v[sl] - LO) * SCALE
        acc_v[sl] = jnp.zeros((L,), jnp.float32)

    @plsc.parallel_loop(0, K // L, unroll=UNROLL)
    def _inith(i):
        sl = pl.ds(i * L, L)
        h_v[sl] = jnp.zeros((L,), jnp.int32)

    def process_pair(ga_v, gb_v):
        @plsc.parallel_loop(0, D // L, unroll=UNROLL)
        def _pass1(i):
            sl = pl.ds(i * L, L)
            xs = xs_v[sl]
            ta = xs + ga_v[sl] * (SIGMA * SCALE)
            tb = xs + gb_v[sl] * (SIGMA * SCALE)
            bia = plsc.bitcast(jnp.minimum(
                plsc.bitcast(ta.astype(jnp.int32), jnp.uint32),
                jnp.uint32(K - 1)), jnp.int32)
            bib = plsc.bitcast(jnp.minimum(
                plsc.bitcast(tb.astype(jnp.int32), jnp.uint32),
                jnp.uint32(K - 1)), jnp.int32)
            ba_v[sl] = jnp.bitwise_or(bia, lax.shift_left(bib, 16))
            plsc.addupdate_scatter(h_v, [bia], jnp.ones((L,), jnp.int32))
            plsc.addupdate_scatter(h_v, [bib],
                                   jnp.full((L,), 65536, jnp.int32))

        @plsc.parallel_loop(0, K // L, unroll=UNROLL,
                            carry=(jnp.int32(0), jnp.int32(0)))
        def _cum(i, carry):
            ca, cb = carry
            sl = pl.ds(i * L, L)
            h = h_v[sl]
            h_v[sl] = jnp.zeros((L,), jnp.int32)
            ha = jnp.bitwise_and(h, 0xFFFF)
            hb = lax.shift_right_logical(h, 16)
            inca = plsc.cumsum(ha) + ca
            incb = plsc.cumsum(hb) + cb
            vala_v[sl] = (inca.astype(jnp.float32)
                          - 0.5 * ha.astype(jnp.float32) - 0.5)
            valb_v[sl] = (incb.astype(jnp.float32)
                          - 0.5 * hb.astype(jnp.float32) - 0.5)
            return (ca + jnp.sum(ha), cb + jnp.sum(hb))

        @plsc.parallel_loop(0, D // L, unroll=UNROLL)
        def _pass2(i):
            sl = pl.ds(i * L, L)
            w = ba_v[sl]
            ra = plsc.load_gather(vala_v, [jnp.bitwise_and(w, 0xFFFF)])
            rb = plsc.load_gather(valb_v, [lax.shift_right_logical(w, 16)])
            plsc.addupdate(acc_v.at[sl], ra + rb)

    def row(s):
        return s * B + wid

    def fetch(pair, ga, gb, sa, sb):
        pltpu.async_copy(g_hbm.at[row(pair * 2)], ga, sa)
        pltpu.async_copy(g_hbm.at[row(pair * 2 + 1)], gb, sb)

    def wait(pair, ga, gb, sa, sb):
        pltpu.make_async_copy(g_hbm.at[row(pair * 2)], ga, sa).wait()
        pltpu.make_async_copy(g_hbm.at[row(pair * 2 + 1)], gb, sb).wait()

    NP = NUM_SAMPLES // 2
    fetch(0, g0a_v, g0b_v, s0a, s0b)

    def pair2_body(q, _):
        p0 = q * 2
        fetch(p0 + 1, g1a_v, g1b_v, s1a, s1b)
        wait(p0, g0a_v, g0b_v, s0a, s0b)
        process_pair(g0a_v, g0b_v)
        nxt = jnp.minimum(p0 + 2, NP - 1)
        fetch(nxt, g0a_v, g0b_v, s0a, s0b)
        wait(p0 + 1, g1a_v, g1b_v, s1a, s1b)
        process_pair(g1a_v, g1b_v)
        return 0

    lax.fori_loop(0, NP // 2, pair2_body, 0)
    wait(NP - 1, g0a_v, g0b_v, s0a, s0b)

    @plsc.parallel_loop(0, D // L, unroll=UNROLL)
    def _fin(i):
        sl = pl.ds(i * L, L)
        acc_v[sl] = acc_v[sl] * (1.0 / NUM_SAMPLES)
    pltpu.sync_copy(acc_v, out_hbm.at[wid])


def kernel(inputs, gumbel_noise):
    noise2d = gumbel_noise.reshape(NUM_SAMPLES * B, D)
    mesh = plsc.VectorSubcoreMesh(core_axis_name="c", subcore_axis_name="s")
    run = functools.partial(
        pl.kernel,
        out_type=jax.ShapeDtypeStruct((B, D), jnp.float32),
        mesh=mesh,
        compiler_params=pltpu.CompilerParams(needs_layout_passes=False),
        scratch_types=[
            pltpu.VMEM((D,), jnp.float32),
            pltpu.VMEM((D,), jnp.float32),
            pltpu.VMEM((D,), jnp.float32),
            pltpu.VMEM((D,), jnp.float32),
            pltpu.VMEM((D,), jnp.float32),
            pltpu.VMEM((D,), jnp.int32),
            pltpu.VMEM((K,), jnp.int32),
            pltpu.VMEM((K,), jnp.float32),
            pltpu.VMEM((K,), jnp.float32),
            pltpu.VMEM((D,), jnp.float32),
            pltpu.SemaphoreType.DMA,
            pltpu.SemaphoreType.DMA,
            pltpu.SemaphoreType.DMA,
            pltpu.SemaphoreType.DMA,
        ],
    )(_rank_mean_kernel)
    return run(inputs, noise2d)


# speedup vs baseline: 1.1095x; 1.0058x over previous
import functools

import jax
import jax.numpy as jnp
from jax import lax
from jax.experimental import pallas as pl
from jax.experimental.pallas import tpu as pltpu, tpu_sc as plsc

NUM_SAMPLES = 128
B = 32
D = 4096
SIGMA = 0.1

K = 2048
LO = -9.0
HI = 9.0
SCALE = K / (HI - LO)

L = 16
NC = 2
UNROLL = 8


def _rank_mean_kernel(x_hbm, g_hbm, out_hbm, xs_v, g0a_v, g0b_v, g1a_v,
                      g1b_v, ba_v, h_v, vala_v, valb_v, acc_v,
                      s0a, s0b, s1a, s1b):
    wid = lax.axis_index("s") * NC + lax.axis_index("c")

    pltpu.sync_copy(x_hbm.at[wid], xs_v)

    @plsc.parallel_loop(0, D // L, unroll=UNROLL)
    def _init(i):
        sl = pl.ds(i * L, L)
        xs_v[sl] = (xs_v[sl] - LO) * SCALE
        acc_v[sl] = jnp.zeros((L,), jnp.float32)

    @plsc.parallel_loop(0, K // L, unroll=UNROLL)
    def _inith(i):
        sl = pl.ds(i * L, L)
        h_v[sl] = jnp.zeros((L,), jnp.int32)

    def process_pair(ga_v, gb_v):
        @plsc.parallel_loop(0, D // L, unroll=UNROLL)
        def _pass1(i):
            sl = pl.ds(i * L, L)
            xs = xs_v[sl]
            ta = xs + ga_v[sl] * (SIGMA * SCALE)
            tb = xs + gb_v[sl] * (SIGMA * SCALE)
            bia = plsc.bitcast(jnp.minimum(
                plsc.bitcast(ta.astype(jnp.int32), jnp.uint32),
                jnp.uint32(K - 1)), jnp.int32)
            bib = plsc.bitcast(jnp.minimum(
                plsc.bitcast(tb.astype(jnp.int32), jnp.uint32),
                jnp.uint32(K - 1)), jnp.int32)
            ba_v[sl] = jnp.bitwise_or(bia, lax.shift_left(bib, 16))
            plsc.addupdate_scatter(h_v, [bia], jnp.ones((L,), jnp.int32))
            plsc.addupdate_scatter(h_v, [bib],
                                   jnp.full((L,), 65536, jnp.int32))

        @plsc.parallel_loop(0, K // L, unroll=4,
                            carry=(jnp.int32(0), jnp.int32(0)))
        def _cum(i, carry):
            ca, cb = carry
            sl = pl.ds(i * L, L)
            h = h_v[sl]
            h_v[sl] = jnp.zeros((L,), jnp.int32)
            ha = jnp.bitwise_and(h, 0xFFFF)
            hb = lax.shift_right_logical(h, 16)
            inca = plsc.cumsum(ha) + ca
            incb = plsc.cumsum(hb) + cb
            vala_v[sl] = (inca.astype(jnp.float32)
                          - 0.5 * ha.astype(jnp.float32) - 0.5)
            valb_v[sl] = (incb.astype(jnp.float32)
                          - 0.5 * hb.astype(jnp.float32) - 0.5)
            return (ca + jnp.sum(ha), cb + jnp.sum(hb))

        @plsc.parallel_loop(0, D // L, unroll=UNROLL)
        def _pass2(i):
            sl = pl.ds(i * L, L)
            w = ba_v[sl]
            ra = plsc.load_gather(vala_v, [jnp.bitwise_and(w, 0xFFFF)])
            rb = plsc.load_gather(valb_v, [lax.shift_right_logical(w, 16)])
            plsc.addupdate(acc_v.at[sl], ra + rb)

    def row(s):
        return s * B + wid

    def fetch(pair, ga, gb, sa, sb):
        pltpu.async_copy(g_hbm.at[row(pair * 2)], ga, sa)
        pltpu.async_copy(g_hbm.at[row(pair * 2 + 1)], gb, sb)

    def wait(pair, ga, gb, sa, sb):
        pltpu.make_async_copy(g_hbm.at[row(pair * 2)], ga, sa).wait()
        pltpu.make_async_copy(g_hbm.at[row(pair * 2 + 1)], gb, sb).wait()

    NP = NUM_SAMPLES // 2
    fetch(0, g0a_v, g0b_v, s0a, s0b)

    def pair2_body(q, _):
        p0 = q * 2
        fetch(p0 + 1, g1a_v, g1b_v, s1a, s1b)
        wait(p0, g0a_v, g0b_v, s0a, s0b)
        process_pair(g0a_v, g0b_v)
        nxt = jnp.minimum(p0 + 2, NP - 1)
        fetch(nxt, g0a_v, g0b_v, s0a, s0b)
        wait(p0 + 1, g1a_v, g1b_v, s1a, s1b)
        process_pair(g1a_v, g1b_v)
        return 0

    lax.fori_loop(0, NP // 2, pair2_body, 0)
    wait(NP - 1, g0a_v, g0b_v, s0a, s0b)

    @plsc.parallel_loop(0, D // L, unroll=UNROLL)
    def _fin(i):
        sl = pl.ds(i * L, L)
        acc_v[sl] = acc_v[sl] * (1.0 / NUM_SAMPLES)
    pltpu.sync_copy(acc_v, out_hbm.at[wid])


def kernel(inputs, gumbel_noise):
    noise2d = gumbel_noise.reshape(NUM_SAMPLES * B, D)
    mesh = plsc.VectorSubcoreMesh(core_axis_name="c", subcore_axis_name="s")
    run = functools.partial(
        pl.kernel,
        out_type=jax.ShapeDtypeStruct((B, D), jnp.float32),
        mesh=mesh,
        compiler_params=pltpu.CompilerParams(needs_layout_passes=False),
        scratch_types=[
            pltpu.VMEM((D,), jnp.float32),
            pltpu.VMEM((D,), jnp.float32),
            pltpu.VMEM((D,), jnp.float32),
            pltpu.VMEM((D,), jnp.float32),
            pltpu.VMEM((D,), jnp.float32),
            pltpu.VMEM((D,), jnp.int32),
            pltpu.VMEM((K,), jnp.int32),
            pltpu.VMEM((K,), jnp.float32),
            pltpu.VMEM((K,), jnp.float32),
            pltpu.VMEM((D,), jnp.float32),
            pltpu.SemaphoreType.DMA,
            pltpu.SemaphoreType.DMA,
            pltpu.SemaphoreType.DMA,
            pltpu.SemaphoreType.DMA,
        ],
    )(_rank_mean_kernel)
    return run(inputs, noise2d)
